# baseline (speedup 1.0000x reference)
import functools

import jax
import jax.numpy as jnp
from jax import lax
from jax.experimental import pallas as pl
from jax.experimental.pallas import tpu as pltpu
from jax.experimental.pallas import tpu_sc as plsc

_EMBED = 128
_CHUNK = 128


def _gather_sc(table, idx4):
    _, ns, n_chunks, _ = idx4.shape
    n_per_w = n_chunks * _CHUNK
    batch = ns * n_per_w
    mesh = plsc.VectorSubcoreMesh(core_axis_name="c", subcore_axis_name="s")

    @functools.partial(
        pl.kernel,
        mesh=mesh,
        out_type=jax.ShapeDtypeStruct((2, batch, _EMBED), jnp.float32),
        scratch_types=[
            pltpu.VMEM((n_chunks, _CHUNK), jnp.int32),
            pltpu.VMEM((3, _CHUNK, _EMBED), jnp.float32),
            pltpu.SemaphoreType.DMA,
            pltpu.SemaphoreType.DMA,
            pltpu.SemaphoreType.DMA,
            pltpu.SemaphoreType.DMA,
            pltpu.SemaphoreType.DMA,
            pltpu.SemaphoreType.DMA,
        ],
    )
    def gather_kernel(
        table_hbm, idx_hbm, out_hbm, idx_v, rows_v,
        gsem0, gsem1, gsem2, wsem0, wsem1, wsem2,
    ):
        half = lax.axis_index("c")
        sub = lax.axis_index("s")
        base = sub * n_per_w
        pltpu.sync_copy(idx_hbm.at[half, sub], idx_v)
        gsems = (gsem0, gsem1, gsem2)
        wsems = (wsem0, wsem1, wsem2)
        nbuf = 3

        def gather(j, b):
            return pltpu.async_copy(
                table_hbm.at[idx_v.at[j]], rows_v.at[b], gsems[b]
            )

        def writeback(j, b):
            return pltpu.async_copy(
                rows_v.at[b],
                out_hbm.at[half, pl.ds(base + j * _CHUNK, _CHUNK)],
                wsems[b],
            )

        g, w = {}, {}
        for j in range(min(nbuf - 1, n_chunks)):
            g[j] = gather(j, j % nbuf)
        for j in range(n_chunks):
            b = j % nbuf
            g.pop(j).wait()
            w[b] = writeback(j, b)
            nj = j + nbuf - 1
            if nj < n_chunks:
                nb = nj % nbuf
                if nb in w:
                    w.pop(nb).wait()
                g[nj] = gather(nj, nb)
        for b in list(w):
            w.pop(b).wait()

    return gather_kernel(table, idx4)


def _mlp_body(o_ref, d_ref, w1a_ref, w1b_ref, b1_ref, w2_ref, b2_ref, o_out):
    h = jnp.dot(o_ref[0], w1a_ref[...], preferred_element_type=jnp.float32)
    h = h + jnp.dot(d_ref[0], w1b_ref[...], preferred_element_type=jnp.float32)
    h = h + b1_ref[...]
    h = jnp.where(h > 0, h, jnp.exp(jnp.minimum(h, 0.0)) - 1.0)
    w2 = w2_ref[...]
    rows = [
        lax.dot_general(
            w2,
            h[i * 128:(i + 1) * 128, :],
            (((0,), (1,)), ((), ())),
            preferred_element_type=jnp.float32,
        )
        for i in range(h.shape[0] // 128)
    ]
    o_out[...] = jnp.concatenate(rows, axis=0) + b2_ref[0, 0]


def _mlp_tc(rows3, w1a, w1b, b1p, w2p, b2):
    bsz = rows3.shape[1]
    bm = 2048
    grid = (bsz // bm,)
    return pl.pallas_call(
        _mlp_body,
        grid=grid,
        in_specs=[
            pl.BlockSpec((1, bm, _EMBED), lambda i: (0, i, 0)),
            pl.BlockSpec((1, bm, _EMBED), lambda i: (1, i, 0)),
            pl.BlockSpec(w1a.shape, lambda i: (0, 0)),
            pl.BlockSpec(w1b.shape, lambda i: (0, 0)),
            pl.BlockSpec(b1p.shape, lambda i: (0, 0)),
            pl.BlockSpec(w2p.shape, lambda i: (0, 0)),
            pl.BlockSpec(b2.shape, lambda i: (0, 0)),
        ],
        out_specs=pl.BlockSpec((bm // 128, 128), lambda i: (i, 0)),
        out_shape=jax.ShapeDtypeStruct((bsz // 128, 128), jnp.float32),
    )(rows3, rows3, w1a, w1b, b1p, w2p, b2)


def kernel(input, table, W1, b1, W2, b2):
    batch = input.shape[0]
    info = plsc.get_sparse_core_info()
    ns = info.num_subcores
    pad = 128 - W1.shape[1]
    w1a = jnp.pad(W1[:_EMBED], ((0, 0), (0, pad)))
    w1b = jnp.pad(W1[_EMBED:], ((0, 0), (0, pad)))
    b1p = jnp.pad(b1, (0, pad)).reshape(1, 128)
    w2p = jnp.pad(W2, ((0, pad), (0, 0)))
    b2r = b2.reshape(1, 1)

    nsplit = 2
    half = batch // nsplit
    idx5 = input.T.reshape(2, nsplit, ns, half // (ns * _CHUNK), _CHUNK)
    idx5 = idx5.transpose(1, 0, 2, 3, 4)
    outs = []
    for k in range(nsplit):
        rows3 = _gather_sc(table, idx5[k])
        outs.append(_mlp_tc(rows3, w1a, w1b, b1p, w2p, b2r))
    return jnp.concatenate(outs, axis=0).reshape(batch, 1)


# speedup vs baseline: 1.1007x; 1.1007x over previous
import functools

import jax
import jax.numpy as jnp
from jax import lax
from jax.experimental import pallas as pl
from jax.experimental.pallas import tpu as pltpu
from jax.experimental.pallas import tpu_sc as plsc

_EMBED = 128
_CHUNK = 128


def _gather_sc(table, idx4):
    _, ns, n_chunks, _ = idx4.shape
    n_per_w = n_chunks * _CHUNK
    batch = ns * n_per_w
    mesh = plsc.VectorSubcoreMesh(core_axis_name="c", subcore_axis_name="s")

    @functools.partial(
        pl.kernel,
        mesh=mesh,
        out_type=jax.ShapeDtypeStruct((2, batch, _EMBED), jnp.float32),
        scratch_types=[
            pltpu.VMEM((n_chunks, _CHUNK), jnp.int32),
            pltpu.VMEM((4, _CHUNK, _EMBED), jnp.float32),
            pltpu.SemaphoreType.DMA,
            pltpu.SemaphoreType.DMA,
            pltpu.SemaphoreType.DMA,
            pltpu.SemaphoreType.DMA,
            pltpu.SemaphoreType.DMA,
            pltpu.SemaphoreType.DMA,
            pltpu.SemaphoreType.DMA,
            pltpu.SemaphoreType.DMA,
        ],
    )
    def gather_kernel(
        table_hbm, idx_hbm, out_hbm, idx_v, rows_v,
        gsem0, gsem1, gsem2, gsem3, wsem0, wsem1, wsem2, wsem3,
    ):
        half = lax.axis_index("c")
        sub = lax.axis_index("s")
        base = sub * n_per_w
        pltpu.sync_copy(idx_hbm.at[half, sub], idx_v)
        gsems = (gsem0, gsem1, gsem2, gsem3)
        wsems = (wsem0, wsem1, wsem2, wsem3)
        nbuf = 4

        def gather(j, b):
            return pltpu.async_copy(
                table_hbm.at[idx_v.at[j]], rows_v.at[b], gsems[b]
            )

        def writeback(j, b):
            return pltpu.async_copy(
                rows_v.at[b],
                out_hbm.at[half, pl.ds(base + j * _CHUNK, _CHUNK)],
                wsems[b],
            )

        g, w = {}, {}
        for j in range(min(nbuf - 1, n_chunks)):
            g[j] = gather(j, j % nbuf)
        for j in range(n_chunks):
            b = j % nbuf
            g.pop(j).wait()
            w[b] = writeback(j, b)
            nj = j + nbuf - 1
            if nj < n_chunks:
                nb = nj % nbuf
                if nb in w:
                    w.pop(nb).wait()
                g[nj] = gather(nj, nb)
        for b in list(w):
            w.pop(b).wait()

    return gather_kernel(table, idx4)


def _mlp_body(o_ref, d_ref, w1a_ref, w1b_ref, b1_ref, w2_ref, b2_ref, o_out):
    h = jnp.dot(o_ref[0], w1a_ref[...], preferred_element_type=jnp.float32)
    h = h + jnp.dot(d_ref[0], w1b_ref[...], preferred_element_type=jnp.float32)
    h = h + b1_ref[...]
    h = jnp.where(h > 0, h, jnp.exp(jnp.minimum(h, 0.0)) - 1.0)
    w2 = w2_ref[...]
    rows = [
        lax.dot_general(
            w2,
            h[i * 128:(i + 1) * 128, :],
            (((0,), (1,)), ((), ())),
            preferred_element_type=jnp.float32,
        )
        for i in range(h.shape[0] // 128)
    ]
    o_out[...] = jnp.concatenate(rows, axis=0) + b2_ref[0, 0]


def _mlp_tc(rows3, w1a, w1b, b1p, w2p, b2):
    bsz = rows3.shape[1]
    bm = 4096
    grid = (bsz // bm,)
    return pl.pallas_call(
        _mlp_body,
        grid=grid,
        in_specs=[
            pl.BlockSpec((1, bm, _EMBED), lambda i: (0, i, 0)),
            pl.BlockSpec((1, bm, _EMBED), lambda i: (1, i, 0)),
            pl.BlockSpec(w1a.shape, lambda i: (0, 0)),
            pl.BlockSpec(w1b.shape, lambda i: (0, 0)),
            pl.BlockSpec(b1p.shape, lambda i: (0, 0)),
            pl.BlockSpec(w2p.shape, lambda i: (0, 0)),
            pl.BlockSpec(b2.shape, lambda i: (0, 0)),
        ],
        out_specs=pl.BlockSpec((bm // 128, 128), lambda i: (i, 0)),
        out_shape=jax.ShapeDtypeStruct((bsz // 128, 128), jnp.float32),
    )(rows3, rows3, w1a, w1b, b1p, w2p, b2)


def kernel(input, table, W1, b1, W2, b2):
    batch = input.shape[0]
    info = plsc.get_sparse_core_info()
    ns = info.num_subcores
    pad = 128 - W1.shape[1]
    w1a = jnp.pad(W1[:_EMBED], ((0, 0), (0, pad)))
    w1b = jnp.pad(W1[_EMBED:], ((0, 0), (0, pad)))
    b1p = jnp.pad(b1, (0, pad)).reshape(1, 128)
    w2p = jnp.pad(W2, ((0, pad), (0, 0)))
    b2r = b2.reshape(1, 1)

    idx4 = input.T.reshape(2, ns, batch // (ns * _CHUNK), _CHUNK)
    rows3 = _gather_sc(table, idx4)
    out2 = _mlp_tc(rows3, w1a, w1b, b1p, w2p, b2r)
    return out2.reshape(batch, 1)
